# baseline (speedup 1.0000x reference)
import functools

import jax
import jax.numpy as jnp
from jax import lax
from jax.experimental import pallas as pl
from jax.experimental.pallas import tpu as pltpu
from jax.experimental.pallas import tpu_sc as plsc

B, P, X, Y = 1024, 64, 32, 32
M = 4096
F = P * X * Y

NC, NS, L = 2, 16, 16
NW = NC * NS
RPW = B // NW


def _policy_flatten_kernel(x_hbm, p_hbm, cx_hbm, cy_hbm, out_hbm,
                           xrow_v, flat_v, cx_v, cy_v, orow_v):
  wid = lax.axis_index("s") * NC + lax.axis_index("c")

  pltpu.sync_copy(p_hbm, flat_v)
  pltpu.sync_copy(cx_hbm, cx_v)
  pltpu.sync_copy(cy_hbm, cy_v)

  def fold(j, carry):
    sl = pl.ds(j * L, L)
    flat_v[sl] = flat_v[sl] * (X * Y) + cx_v[sl] * Y + cy_v[sl]
    return carry

  lax.fori_loop(0, M // L, fold, 0, unroll=4)

  def row(i, carry):
    b = wid * RPW + i
    pltpu.sync_copy(x_hbm.at[b], xrow_v)

    def gather(j, c):
      sl = pl.ds(j * L, L)
      orow_v[sl] = plsc.load_gather(xrow_v, [flat_v[sl]])
      return c

    lax.fori_loop(0, M // L, gather, 0, unroll=8)
    pltpu.sync_copy(orow_v, out_hbm.at[b])
    return carry

  lax.fori_loop(0, RPW, row, 0)


@jax.jit
def kernel(x, piece_orientation_indices, center_placement_x,
           center_placement_y):
  x2 = x.reshape(B, F)
  run = pl.kernel(
      _policy_flatten_kernel,
      out_type=jax.ShapeDtypeStruct((B, M), jnp.float32),
      mesh=plsc.VectorSubcoreMesh(core_axis_name="c", subcore_axis_name="s"),
      scratch_types=[
          pltpu.VMEM((F,), jnp.float32),
          pltpu.VMEM((M,), jnp.int32),
          pltpu.VMEM((M,), jnp.int32),
          pltpu.VMEM((M,), jnp.int32),
          pltpu.VMEM((M,), jnp.float32),
      ],
      compiler_params=pltpu.CompilerParams(needs_layout_passes=False),
  )
  return run(x2,
             piece_orientation_indices.astype(jnp.int32),
             center_placement_x.astype(jnp.int32),
             center_placement_y.astype(jnp.int32))
